# baseline (speedup 1.0000x reference)
import functools

import jax
import jax.numpy as jnp
from jax import lax
from jax.experimental import pallas as pl
from jax.experimental.pallas import tpu as pltpu
from jax.experimental.pallas import tpu_sc as plsc

_NUM_WORKERS = 32
_CHUNK = 128
_NBUF = 4
_LEAD = 2


@functools.partial(jax.jit, static_argnums=(2, 3))
def _sc_gather(table, idx_flat, n, d):
    b_per_w = n // _NUM_WORKERS
    steps = b_per_w // _CHUNK
    mesh = plsc.VectorSubcoreMesh(core_axis_name="c", subcore_axis_name="s")

    @functools.partial(
        pl.kernel,
        mesh=mesh,
        out_type=jax.ShapeDtypeStruct((n, d), jnp.float32),
        scratch_types=[
            pltpu.VMEM((b_per_w,), jnp.int32),
            pltpu.VMEM((_NBUF, _CHUNK, d), jnp.float32),
            pltpu.SemaphoreType.DMA,
            pltpu.SemaphoreType.DMA,
            pltpu.SemaphoreType.DMA,
            pltpu.SemaphoreType.DMA,
            pltpu.SemaphoreType.DMA,
            pltpu.SemaphoreType.DMA,
            pltpu.SemaphoreType.DMA,
            pltpu.SemaphoreType.DMA,
        ],
    )
    def body(table_hbm, idx_hbm, out_hbm, idx_v, rows_v, *sems):
        gsem = sems[:_NBUF]
        wsem = sems[_NBUF:]
        wid = lax.axis_index("s") * 2 + lax.axis_index("c")
        base = pl.multiple_of(wid * b_per_w, _CHUNK)
        pltpu.sync_copy(idx_hbm.at[pl.ds(base, b_per_w)], idx_v)

        def start_gather(g, b):
            off = pl.multiple_of(g * _CHUNK, _CHUNK)
            pltpu.async_copy(
                table_hbm.at[idx_v.at[pl.ds(off, _CHUNK)]], rows_v.at[b], gsem[b]
            )

        def wait_gather(b):
            pltpu.make_async_copy(
                table_hbm.at[pl.ds(0, _CHUNK)], rows_v.at[b], gsem[b]
            ).wait()

        def start_write(g, b):
            off = pl.multiple_of(g * _CHUNK, _CHUNK)
            pltpu.async_copy(rows_v.at[b], out_hbm.at[pl.ds(base + off, _CHUNK)], wsem[b])

        def wait_write(g, b):
            off = pl.multiple_of(g * _CHUNK, _CHUNK)
            pltpu.make_async_copy(
                rows_v.at[b], out_hbm.at[pl.ds(base + off, _CHUNK)], wsem[b]
            ).wait()

        for b in range(_LEAD):
            start_gather(b, b)

        def outer(i, carry):
            for b in range(_NBUF):
                g = i * _NBUF + b
                bn = (b + _LEAD) % _NBUF
                wait_gather(b)
                start_write(g, b)

                @pl.when(g >= _LEAD)
                def _():
                    wait_write(g - _LEAD, bn)

                @pl.when(g + _LEAD < steps)
                def _():
                    start_gather(g + _LEAD, bn)

            return carry

        lax.fori_loop(0, steps // _NBUF, outer, 0)
        for g in range(steps - _LEAD, steps):
            wait_write(g, g % _NBUF)

    return body(table, idx_flat)


def kernel(token_ids, embedding_matrix):
    b, t = token_ids.shape
    v, d = embedding_matrix.shape
    n = b * t
    idx_flat = token_ids.reshape(n).astype(jnp.int32)
    out = _sc_gather(embedding_matrix, idx_flat, n, d)
    return out.reshape(b, t, d)


# speedup vs baseline: 1.0047x; 1.0047x over previous
import functools

import jax
import jax.numpy as jnp
from jax import lax
from jax.experimental import pallas as pl
from jax.experimental.pallas import tpu as pltpu
from jax.experimental.pallas import tpu_sc as plsc

_NUM_WORKERS = 32
_CHUNK = 128
_NBUF = 6
_LEAD = 3


@functools.partial(jax.jit, static_argnums=(2, 3))
def _sc_gather(table, idx_flat, n, d):
    b_per_w = n // _NUM_WORKERS
    steps = b_per_w // _CHUNK
    mesh = plsc.VectorSubcoreMesh(core_axis_name="c", subcore_axis_name="s")

    @functools.partial(
        pl.kernel,
        mesh=mesh,
        out_type=jax.ShapeDtypeStruct((n, d), jnp.float32),
        scratch_types=[
            pltpu.VMEM((b_per_w,), jnp.int32),
            pltpu.VMEM((_NBUF, _CHUNK, d), jnp.float32),
        ] + [pltpu.SemaphoreType.DMA] * (2 * _NBUF),
    )
    def body(table_hbm, idx_hbm, out_hbm, idx_v, rows_v, *sems):
        gsem = sems[:_NBUF]
        wsem = sems[_NBUF:]
        wid = lax.axis_index("s") * 2 + lax.axis_index("c")
        base = pl.multiple_of(wid * b_per_w, _CHUNK)
        pltpu.sync_copy(idx_hbm.at[pl.ds(base, b_per_w)], idx_v)

        def start_gather(g, b):
            off = pl.multiple_of(g * _CHUNK, _CHUNK)
            pltpu.async_copy(
                table_hbm.at[idx_v.at[pl.ds(off, _CHUNK)]], rows_v.at[b], gsem[b]
            )

        def wait_gather(b):
            pltpu.make_async_copy(
                table_hbm.at[pl.ds(0, _CHUNK)], rows_v.at[b], gsem[b]
            ).wait()

        def start_write(g, b):
            off = pl.multiple_of(g * _CHUNK, _CHUNK)
            pltpu.async_copy(rows_v.at[b], out_hbm.at[pl.ds(base + off, _CHUNK)], wsem[b])

        def wait_write(g, b):
            off = pl.multiple_of(g * _CHUNK, _CHUNK)
            pltpu.make_async_copy(
                rows_v.at[b], out_hbm.at[pl.ds(base + off, _CHUNK)], wsem[b]
            ).wait()

        for b in range(_LEAD):
            start_gather(b, b)

        def visit(g, b):
            bn = (b + _LEAD) % _NBUF
            wait_gather(b)
            start_write(g, b)

            @pl.when(g >= _LEAD)
            def _():
                wait_write(g - _LEAD, bn)

            @pl.when(g + _LEAD < steps)
            def _():
                start_gather(g + _LEAD, bn)

        def outer(i, carry):
            for b in range(_NBUF):
                visit(i * _NBUF + b, b)
            return carry

        full = steps // _NBUF
        lax.fori_loop(0, full, outer, 0)
        for g in range(full * _NBUF, steps):
            visit(g, g % _NBUF)
        for g in range(steps - _LEAD, steps):
            wait_write(g, g % _NBUF)

    return body(table, idx_flat)


def kernel(token_ids, embedding_matrix):
    b, t = token_ids.shape
    v, d = embedding_matrix.shape
    n = b * t
    idx_flat = token_ids.reshape(n).astype(jnp.int32)
    out = _sc_gather(embedding_matrix, idx_flat, n, d)
    return out.reshape(b, t, d)
